# baseline (speedup 1.0000x reference)
import jax
import jax.numpy as jnp
from jax.experimental import pallas as pl
from jax.experimental.pallas import tpu as pltpu

_EPS = 1e-12
_K = 16
_BJ = 256
_TW = 128


def _chebyshev_apply(c_ref, g, g2, v, ls):
    bc = c_ref[_K] * v
    bp = jnp.zeros_like(v)
    for k in range(_K - 1, 0, -1):
        bn = c_ref[k] * v + jnp.dot(
            g2, bc, preferred_element_type=jnp.float32) * ls - bp
        bp = bc
        bc = bn
    return c_ref[0] * v + jnp.dot(
        g, bc, preferred_element_type=jnp.float32) * ls - bp


def _fused_body(c_ref, x_ref, g_ref, g2_ref, ls_ref, qs_ref, lst_ref,
                qst_ref, out_ref):
    j = pl.program_id(0)
    x = x_ref[...]
    g = g_ref[...]
    g2 = g2_ref[...]
    qs = qs_ref[...]

    v = jnp.dot(x, qs, preferred_element_type=jnp.float32)
    y = _chebyshev_apply(c_ref, g, g2, v, ls_ref[...])
    z = jax.lax.dot_general(
        y, qs, (((1,), (1,)), ((), ())), preferred_element_type=jnp.float32)

    @pl.when(j == 0)
    def _init():
        out_ref[...] = jnp.zeros_like(out_ref)

    out_ref[...] += z

    @pl.when(j == pl.num_programs(0) - 1)
    def _tail():
        qst = qst_ref[...]
        vt = jnp.dot(x, qst, preferred_element_type=jnp.float32)
        yt = _chebyshev_apply(c_ref, g, g2, vt, lst_ref[...])
        zt = jax.lax.dot_general(
            yt, qst, (((1,), (1,)), ((), ())),
            preferred_element_type=jnp.float32)
        out_ref[...] += zt


def kernel(X, F, Q_S, Lambda_S, gamma):
    m, n = X.shape
    FF = F.T @ F
    G = (FF / (jnp.linalg.norm(FF) + _EPS)).astype(jnp.float32)

    gam = jnp.asarray(gamma, jnp.float32)
    a = 1.0 / gam
    s = jnp.sqrt(a * a - 1.0)
    q = a - s
    scale = 2.0 / (gam * s)
    ks = jnp.arange(_K + 1, dtype=jnp.float32)
    c = scale * q ** ks
    c = c.at[0].multiply(0.5)

    ls_row = Lambda_S.astype(jnp.float32).reshape(1, n)
    nj = n // _BJ
    n_main = nj * _BJ
    rem = n - n_main
    qs_tail = jnp.pad(Q_S[:, n_main:], ((0, 0), (0, _TW - rem)))
    ls_tail = jnp.pad(ls_row[:, n_main:], ((0, 0), (0, _TW - rem)))

    Z = pl.pallas_call(
        _fused_body,
        grid=(nj,),
        in_specs=[
            pl.BlockSpec(memory_space=pltpu.SMEM),
            pl.BlockSpec((m, n), lambda j: (0, 0)),
            pl.BlockSpec((m, m), lambda j: (0, 0)),
            pl.BlockSpec((m, m), lambda j: (0, 0)),
            pl.BlockSpec((1, _BJ), lambda j: (0, j)),
            pl.BlockSpec((n, _BJ), lambda j: (0, j)),
            pl.BlockSpec((1, _TW), lambda j: (0, 0)),
            pl.BlockSpec((n, _TW), lambda j: (0, 0)),
        ],
        out_specs=pl.BlockSpec((m, n), lambda j: (0, 0)),
        out_shape=jax.ShapeDtypeStruct((m, n), jnp.float32),
    )(c, X, G, 2.0 * G, ls_row, Q_S, ls_tail, qs_tail)
    return Z


# speedup vs baseline: 1.4072x; 1.4072x over previous
import jax
import jax.numpy as jnp
from jax.experimental import pallas as pl
from jax.experimental.pallas import tpu as pltpu

_EPS = 1e-12
_K = 16
_BJ = 512


def _chebyshev_apply(c_ref, g, g2, v, ls):
    bc = c_ref[_K] * v
    bp = jnp.zeros_like(v)
    for k in range(_K - 1, 0, -1):
        bn = c_ref[k] * v + jnp.dot(
            g2, bc, preferred_element_type=jnp.float32) * ls - bp
        bp = bc
        bc = bn
    return c_ref[0] * v + jnp.dot(
        g, bc, preferred_element_type=jnp.float32) * ls - bp


def _fused_body(c_ref, x_ref, g_ref, g2_ref, ls_ref, qs_ref, out_ref):
    j = pl.program_id(0)
    qs = qs_ref[...]

    v = jnp.dot(x_ref[...], qs, preferred_element_type=jnp.float32)
    y = _chebyshev_apply(c_ref, g_ref[...], g2_ref[...], v, ls_ref[...])
    z = jax.lax.dot_general(
        y, qs, (((1,), (1,)), ((), ())), preferred_element_type=jnp.float32)

    @pl.when(j == 0)
    def _init():
        out_ref[...] = jnp.zeros_like(out_ref)

    out_ref[...] += z


def kernel(X, F, Q_S, Lambda_S, gamma):
    m, n = X.shape
    FF = F.T @ F
    G = (FF / (jnp.linalg.norm(FF) + _EPS)).astype(jnp.float32)

    gam = jnp.asarray(gamma, jnp.float32)
    a = 1.0 / gam
    s = jnp.sqrt(a * a - 1.0)
    q = a - s
    scale = 2.0 / (gam * s)
    ks = jnp.arange(_K + 1, dtype=jnp.float32)
    c = scale * q ** ks
    c = c.at[0].multiply(0.5)

    ls_row = Lambda_S.astype(jnp.float32).reshape(1, n)
    nj = n // _BJ
    n_main = nj * _BJ

    Z = pl.pallas_call(
        _fused_body,
        grid=(nj,),
        in_specs=[
            pl.BlockSpec(memory_space=pltpu.SMEM),
            pl.BlockSpec((m, n), lambda j: (0, 0)),
            pl.BlockSpec((m, m), lambda j: (0, 0)),
            pl.BlockSpec((m, m), lambda j: (0, 0)),
            pl.BlockSpec((1, _BJ), lambda j: (0, j)),
            pl.BlockSpec((n, _BJ), lambda j: (0, j)),
        ],
        out_specs=pl.BlockSpec((m, n), lambda j: (0, 0)),
        out_shape=jax.ShapeDtypeStruct((m, n), jnp.float32),
        compiler_params=pltpu.CompilerParams(
            dimension_semantics=("arbitrary",),
            vmem_limit_bytes=100 * 1024 * 1024,
        ),
    )(c, X, G, 2.0 * G, ls_row, Q_S)

    qs_t = Q_S[:, n_main:]
    ls_t = ls_row[:, n_main:]
    vt = X @ qs_t
    bc = c[_K] * vt
    bp = jnp.zeros_like(vt)
    for k in range(_K - 1, 0, -1):
        bn = c[k] * vt + 2.0 * (G @ bc) * ls_t - bp
        bp = bc
        bc = bn
    yt = c[0] * vt + (G @ bc) * ls_t - bp
    return Z + yt @ qs_t.T


# speedup vs baseline: 1.5459x; 1.0985x over previous
import jax
import jax.numpy as jnp
from jax.experimental import pallas as pl
from jax.experimental.pallas import tpu as pltpu

_EPS = 1e-12
_K = 16
_BJ = 512


def _chebyshev_apply(c_ref, g, g2, v, ls):
    bc = c_ref[_K] * v
    bp = jnp.zeros_like(v)
    for k in range(_K - 1, 0, -1):
        bn = c_ref[k] * v + jnp.dot(
            g2, bc, preferred_element_type=jnp.float32) * ls - bp
        bp = bc
        bc = bn
    return c_ref[0] * v + jnp.dot(
        g, bc, preferred_element_type=jnp.float32) * ls - bp


def _fused_body(c_ref, x_ref, g_ref, g2_ref, ls_ref, qs_ref, out_ref):
    j = pl.program_id(0)
    qs = qs_ref[...]

    v = jnp.dot(x_ref[...], qs, preferred_element_type=jnp.float32)
    y = _chebyshev_apply(c_ref, g_ref[...], g2_ref[...], v, ls_ref[...])
    z = jax.lax.dot_general(
        y, qs, (((1,), (1,)), ((), ())), preferred_element_type=jnp.float32)

    @pl.when(j == 0)
    def _init():
        out_ref[...] = jnp.zeros_like(out_ref)

    out_ref[...] += z


def _tail_body(c_ref, x_ref, g_ref, g2_ref, ls_ref, qs_ref, zin_ref, out_ref):
    n = x_ref.shape[1]
    bj = qs_ref.shape[1]
    rem = n - (n // bj) * bj
    col = jax.lax.broadcasted_iota(jnp.int32, (1, bj), 1)
    valid = col < rem
    qs = jnp.where(valid, qs_ref[...], 0.0)
    ls = jnp.where(valid, ls_ref[...], 0.0)
    v = jnp.dot(x_ref[...], qs, preferred_element_type=jnp.float32)
    y = _chebyshev_apply(c_ref, g_ref[...], g2_ref[...], v, ls)
    zt = jax.lax.dot_general(
        y, qs, (((1,), (1,)), ((), ())), preferred_element_type=jnp.float32)
    out_ref[...] = zin_ref[...] + zt


def kernel(X, F, Q_S, Lambda_S, gamma):
    m, n = X.shape
    FF = F.T @ F
    G = (FF / (jnp.linalg.norm(FF) + _EPS)).astype(jnp.float32)

    gam = jnp.asarray(gamma, jnp.float32)
    a = 1.0 / gam
    s = jnp.sqrt(a * a - 1.0)
    q = a - s
    scale = 2.0 / (gam * s)
    ks = jnp.arange(_K + 1, dtype=jnp.float32)
    c = scale * q ** ks
    c = c.at[0].multiply(0.5)

    ls_row = Lambda_S.astype(jnp.float32).reshape(1, n)
    nj = n // _BJ

    Z_main = pl.pallas_call(
        _fused_body,
        grid=(nj,),
        in_specs=[
            pl.BlockSpec(memory_space=pltpu.SMEM),
            pl.BlockSpec((m, n), lambda j: (0, 0)),
            pl.BlockSpec((m, m), lambda j: (0, 0)),
            pl.BlockSpec((m, m), lambda j: (0, 0)),
            pl.BlockSpec((1, _BJ), lambda j: (0, j)),
            pl.BlockSpec((n, _BJ), lambda j: (0, j)),
        ],
        out_specs=pl.BlockSpec((m, n), lambda j: (0, 0)),
        out_shape=jax.ShapeDtypeStruct((m, n), jnp.float32),
        compiler_params=pltpu.CompilerParams(
            dimension_semantics=("arbitrary",),
            vmem_limit_bytes=100 * 1024 * 1024,
        ),
    )(c, X, G, 2.0 * G, ls_row, Q_S)

    Z = pl.pallas_call(
        _tail_body,
        grid=(1,),
        in_specs=[
            pl.BlockSpec(memory_space=pltpu.SMEM),
            pl.BlockSpec((m, n), lambda j: (0, 0)),
            pl.BlockSpec((m, m), lambda j: (0, 0)),
            pl.BlockSpec((m, m), lambda j: (0, 0)),
            pl.BlockSpec((1, _BJ), lambda j: (0, nj)),
            pl.BlockSpec((n, _BJ), lambda j: (0, nj)),
            pl.BlockSpec((m, n), lambda j: (0, 0)),
        ],
        out_specs=pl.BlockSpec((m, n), lambda j: (0, 0)),
        out_shape=jax.ShapeDtypeStruct((m, n), jnp.float32),
        input_output_aliases={6: 0},
        compiler_params=pltpu.CompilerParams(
            dimension_semantics=("arbitrary",),
            vmem_limit_bytes=100 * 1024 * 1024,
        ),
    )(c, X, G, 2.0 * G, ls_row, Q_S, Z_main)
    return Z


# speedup vs baseline: 1.6200x; 1.0480x over previous
import jax
import jax.numpy as jnp
from jax.experimental import pallas as pl
from jax.experimental.pallas import tpu as pltpu

_EPS = 1e-12
_K = 16
_BJ = 512


def _chebyshev_apply(c_ref, g, g2, v, ls):
    bc = c_ref[_K] * v
    bp = jnp.zeros_like(v)
    for k in range(_K - 1, 0, -1):
        bn = c_ref[k] * v + jnp.dot(
            g2, bc, preferred_element_type=jnp.float32) * ls - bp
        bp = bc
        bc = bn
    return c_ref[0] * v + jnp.dot(
        g, bc, preferred_element_type=jnp.float32) * ls - bp


def _fused_body(c_ref, x_ref, g_ref, g2_ref, ls_ref, qs_ref, out_ref):
    j = pl.program_id(0)
    n = x_ref.shape[1]
    bj = qs_ref.shape[1]
    qs = qs_ref[...]

    col = jax.lax.broadcasted_iota(jnp.int32, (1, bj), 1)
    valid = col < (n - j * bj)
    ls = jnp.where(valid, ls_ref[...], 0.0)

    v = jnp.dot(x_ref[...], qs, preferred_element_type=jnp.float32)
    v = jnp.where(valid, v, 0.0)
    y = _chebyshev_apply(c_ref, g_ref[...], g2_ref[...], v, ls)
    z = jax.lax.dot_general(
        y, qs, (((1,), (1,)), ((), ())), preferred_element_type=jnp.float32)

    @pl.when(j == 0)
    def _init():
        out_ref[...] = jnp.zeros_like(out_ref)

    out_ref[...] += z


def kernel(X, F, Q_S, Lambda_S, gamma):
    m, n = X.shape
    FF = F.T @ F
    G = (FF / (jnp.linalg.norm(FF) + _EPS)).astype(jnp.float32)

    gam = jnp.asarray(gamma, jnp.float32)
    a = 1.0 / gam
    s = jnp.sqrt(a * a - 1.0)
    q = a - s
    scale = 2.0 / (gam * s)
    ks = jnp.arange(_K + 1, dtype=jnp.float32)
    c = scale * q ** ks
    c = c.at[0].multiply(0.5)

    ls_row = Lambda_S.astype(jnp.float32).reshape(1, n)
    nj = pl.cdiv(n, _BJ)

    Z = pl.pallas_call(
        _fused_body,
        grid=(nj,),
        in_specs=[
            pl.BlockSpec(memory_space=pltpu.SMEM),
            pl.BlockSpec((m, n), lambda j: (0, 0)),
            pl.BlockSpec((m, m), lambda j: (0, 0)),
            pl.BlockSpec((m, m), lambda j: (0, 0)),
            pl.BlockSpec((1, _BJ), lambda j: (0, j)),
            pl.BlockSpec((n, _BJ), lambda j: (0, j)),
        ],
        out_specs=pl.BlockSpec((m, n), lambda j: (0, 0)),
        out_shape=jax.ShapeDtypeStruct((m, n), jnp.float32),
        compiler_params=pltpu.CompilerParams(
            dimension_semantics=("arbitrary",),
            vmem_limit_bytes=100 * 1024 * 1024,
        ),
    )(c, X, G, 2.0 * G, ls_row, Q_S)
    return Z
